# baseline (speedup 1.0000x reference)
import functools

import jax
import jax.numpy as jnp
import numpy as np
from jax.experimental import pallas as pl
from jax.experimental.pallas import tpu as pltpu

NEG = -1e9
EPS = 1e-12


def _ln(x, g, b):
    m = jnp.mean(x, axis=-1, keepdims=True)
    v = jnp.mean((x - m) ** 2, axis=-1, keepdims=True)
    return (x - m) / jnp.sqrt(v + EPS) * g + b


def _dot(a, b):
    return jax.lax.dot_general(a, b, (((1,), (0,)), ((), ())),
                               preferred_element_type=jnp.float32)


def _dot_t(a, b):
    return jax.lax.dot_general(a, b, (((1,), (1,)), ((), ())),
                               preferred_element_type=jnp.float32)


def _moe_kernel(tag_ref, wm_ref, bm_ref, wg_ref, we_ref, be_ref, out_ref, *, E, DP):
    x = _dot(tag_ref[...], wm_ref[...]) + bm_ref[...]
    logits = _dot(x, wg_ref[...])
    iota = jax.lax.broadcasted_iota(jnp.int32, logits.shape, 1)
    m1 = jnp.max(logits, axis=-1, keepdims=True)
    i1 = jnp.min(jnp.where(logits == m1, iota, E), axis=-1, keepdims=True)
    oh1 = (iota == i1)
    masked = jnp.where(oh1, -1e30, logits)
    m2 = jnp.max(masked, axis=-1, keepdims=True)
    i2 = jnp.min(jnp.where(masked == m2, iota, E), axis=-1, keepdims=True)
    oh2 = (iota == i2)
    e2 = jnp.exp(m2 - m1)
    g1 = 1.0 / (1.0 + e2)
    g2 = e2 / (1.0 + e2)
    w = g1 * oh1.astype(jnp.float32) + g2 * oh2.astype(jnp.float32)
    eo = jnp.maximum(_dot(x, we_ref[...]) + be_ref[...], 0.0)
    acc = x
    for e in range(E):
        acc = acc + w[:, e:e + 1] * eo[:, e * DP:(e + 1) * DP]
    out_ref[...] = acc


def _pattn_kernel(pr_ref, wq_ref, bq_ref, wk_ref, bk_ref, wv_ref, bv_ref,
                  wo_ref, bo_ref, gpr_ref, bpr_ref, wp_ref, bp_ref,
                  gh_ref, bh_ref, out_ref, *, NH, HD, QBLK):
    qi = pl.program_id(1)
    pr = pr_ref[0]
    S, DP = pr.shape
    normed = _ln(pr, gpr_ref[...], bpr_ref[...])
    kh = _dot(normed, wk_ref[...]) + bk_ref[...]
    vh = _dot(normed, wv_ref[...]) + bv_ref[...]
    q0 = qi * QBLK
    pr_q = pr_ref[0, pl.ds(q0, QBLK), :]
    nq = _ln(pr_q, gpr_ref[...], bpr_ref[...])
    qh = _dot(nq, wq_ref[...]) + bq_ref[...]

    lane = jax.lax.broadcasted_iota(jnp.int32, (1, DP), 1)
    row = jax.lax.broadcasted_iota(jnp.int32, (QBLK, S), 0) + q0
    col = jax.lax.broadcasted_iota(jnp.int32, (QBLK, S), 1)
    causal = col > row
    scale = 1.0 / np.sqrt(float(HD))

    def body(h, acc):
        hm = (lane // HD == h).astype(jnp.float32)
        s = _dot_t(qh * hm, kh) * scale
        s = jnp.where(causal, NEG, s)
        m = jnp.max(s, axis=-1, keepdims=True)
        e = jnp.exp(s - m)
        p = e / jnp.sum(e, axis=-1, keepdims=True)
        return acc + _dot(p, vh * hm)

    acc = jax.lax.fori_loop(0, NH, body, jnp.zeros((QBLK, DP), jnp.float32))
    o = _dot(acc, wo_ref[...]) + bo_ref[...] + pr_q
    pr768 = _dot(o, wp_ref[...]) + bp_ref[...]
    out_ref[0] = _ln(pr768, gh_ref[...], bh_ref[...])


def _kvproj_kernel(np_ref, wk_ref, bk_ref, wv_ref, bv_ref, k_ref, v_ref):
    x = np_ref[0]
    k_ref[0] = _dot(x, wk_ref[...]) + bk_ref[...]
    v_ref[0] = _dot(x, wv_ref[...]) + bv_ref[...]


def _fattn_kernel(hid_ref, k_ref, v_ref, wq_ref, bq_ref, wo_ref, bo_ref,
                  gf_ref, bf_ref, out_ref, *, NH, HD, QBLK):
    qi = pl.program_id(1)
    h = pl.program_id(2)
    hid = hid_ref[0]
    q = _dot(hid, wq_ref[0]) + bq_ref[0]
    k = k_ref[0, 0]
    S = k.shape[0]
    s = _dot_t(q, k) * (1.0 / np.sqrt(float(HD)))
    row = jax.lax.broadcasted_iota(jnp.int32, (QBLK, S), 0) + qi * QBLK
    col = jax.lax.broadcasted_iota(jnp.int32, (QBLK, S), 1)
    s = jnp.where(col > row, NEG, s)
    m = jnp.max(s, axis=-1, keepdims=True)
    e = jnp.exp(s - m)
    p = e / jnp.sum(e, axis=-1, keepdims=True)
    o = _dot(p, v_ref[0, 0])
    part = _dot(o, wo_ref[0])

    @pl.when(h == 0)
    def _():
        out_ref[0] = part + bo_ref[...] + hid

    @pl.when(h > 0)
    def _():
        out_ref[0] = out_ref[0] + part

    @pl.when(h == NH - 1)
    def _():
        out_ref[0] = _ln(out_ref[0], gf_ref[...], bf_ref[...])


def _sattn_kernel(x_ref, f_ref, wq_ref, bq_ref, wk_ref, bk_ref, wv_ref,
                  bv_ref, wo_ref, bo_ref, go_ref, bo2_ref, out_ref,
                  *, NH, HD):
    h = pl.program_id(2)
    x = x_ref[0]
    f = f_ref[0]
    q = _dot(x, wq_ref[0]) + bq_ref[0]
    k = _dot(f, wk_ref[0]) + bk_ref[0]
    v = _dot(f, wv_ref[0]) + bv_ref[0]
    s = _dot_t(q, k) * (1.0 / np.sqrt(float(HD)))
    m = jnp.max(s, axis=-1, keepdims=True)
    e = jnp.exp(s - m)
    p = e / jnp.sum(e, axis=-1, keepdims=True)
    o = _dot(p, v)
    part = _dot(o, wo_ref[0])

    @pl.when(h == 0)
    def _():
        out_ref[0] = part + bo_ref[...] + x

    @pl.when(h > 0)
    def _():
        out_ref[0] = out_ref[0] + part

    @pl.when(h == NH - 1)
    def _():
        out_ref[0] = _ln(out_ref[0], go_ref[...], bo2_ref[...])


def _row(x):
    return x.reshape(1, -1)


def kernel(hidden_states, tag_hidden_state, feats, video_ids, params):
    p = params
    B, S, D = hidden_states.shape
    TAG_D = tag_hidden_state.shape[2]
    DP = p['Wm'].shape[1]
    E = p['Wg'].shape[1]
    NH = D // 64
    HD = D // NH
    HDP = DP // NH
    F = feats.shape[1]
    QBLK = 256
    NQ = S // QBLK
    MBLK = 512
    NM = (B * S) // MBLK

    f32 = jnp.float32

    tag2d = tag_hidden_state.reshape(B * S, TAG_D)
    we2 = p['We'].transpose(1, 0, 2).reshape(DP, E * DP)
    be2 = p['be'].reshape(1, E * DP)
    priori = pl.pallas_call(
        functools.partial(_moe_kernel, E=E, DP=DP),
        grid=(NM,),
        in_specs=[
            pl.BlockSpec((MBLK, TAG_D), lambda i: (i, 0)),
            pl.BlockSpec((TAG_D, DP), lambda i: (0, 0)),
            pl.BlockSpec((1, DP), lambda i: (0, 0)),
            pl.BlockSpec((DP, E), lambda i: (0, 0)),
            pl.BlockSpec((DP, E * DP), lambda i: (0, 0)),
            pl.BlockSpec((1, E * DP), lambda i: (0, 0)),
        ],
        out_specs=pl.BlockSpec((MBLK, DP), lambda i: (i, 0)),
        out_shape=jax.ShapeDtypeStruct((B * S, DP), f32),
    )(tag2d, p['Wm'], _row(p['bm']), p['Wg'], we2, be2)
    priori = priori.reshape(B, S, DP)

    pa = p['pattn']
    normed_priori = pl.pallas_call(
        functools.partial(_pattn_kernel, NH=NH, HD=HDP, QBLK=QBLK),
        grid=(B, NQ),
        in_specs=[
            pl.BlockSpec((1, S, DP), lambda b, q: (b, 0, 0)),
            pl.BlockSpec((DP, DP), lambda b, q: (0, 0)),
            pl.BlockSpec((1, DP), lambda b, q: (0, 0)),
            pl.BlockSpec((DP, DP), lambda b, q: (0, 0)),
            pl.BlockSpec((1, DP), lambda b, q: (0, 0)),
            pl.BlockSpec((DP, DP), lambda b, q: (0, 0)),
            pl.BlockSpec((1, DP), lambda b, q: (0, 0)),
            pl.BlockSpec((DP, DP), lambda b, q: (0, 0)),
            pl.BlockSpec((1, DP), lambda b, q: (0, 0)),
            pl.BlockSpec((1, DP), lambda b, q: (0, 0)),
            pl.BlockSpec((1, DP), lambda b, q: (0, 0)),
            pl.BlockSpec((DP, D), lambda b, q: (0, 0)),
            pl.BlockSpec((1, D), lambda b, q: (0, 0)),
            pl.BlockSpec((1, D), lambda b, q: (0, 0)),
            pl.BlockSpec((1, D), lambda b, q: (0, 0)),
        ],
        out_specs=pl.BlockSpec((1, QBLK, D), lambda b, q: (b, q, 0)),
        out_shape=jax.ShapeDtypeStruct((B, S, D), f32),
    )(priori, pa['Wq'], _row(pa['bq']), pa['Wk'], _row(pa['bk']),
      pa['Wv'], _row(pa['bv']), pa['Wo'], _row(pa['bo']),
      _row(p['g_pr']), _row(p['b_pr']), p['Wp'], _row(p['bp']),
      _row(p['g_h']), _row(p['b_h']))

    fa = p['fattn']
    kp, vp = pl.pallas_call(
        _kvproj_kernel,
        grid=(B, NQ),
        in_specs=[
            pl.BlockSpec((1, QBLK, D), lambda b, q: (b, q, 0)),
            pl.BlockSpec((D, D), lambda b, q: (0, 0)),
            pl.BlockSpec((1, D), lambda b, q: (0, 0)),
            pl.BlockSpec((D, D), lambda b, q: (0, 0)),
            pl.BlockSpec((1, D), lambda b, q: (0, 0)),
        ],
        out_specs=[
            pl.BlockSpec((1, QBLK, D), lambda b, q: (b, q, 0)),
            pl.BlockSpec((1, QBLK, D), lambda b, q: (b, q, 0)),
        ],
        out_shape=[jax.ShapeDtypeStruct((B, S, D), f32),
                   jax.ShapeDtypeStruct((B, S, D), f32)],
    )(normed_priori, fa['Wk'], _row(fa['bk']), fa['Wv'], _row(fa['bv']))
    kph = kp.reshape(B, S, NH, HD).transpose(0, 2, 1, 3)
    vph = vp.reshape(B, S, NH, HD).transpose(0, 2, 1, 3)

    wq_f = fa['Wq'].reshape(D, NH, HD).transpose(1, 0, 2)
    bq_f = fa['bq'].reshape(NH, 1, HD)
    wo_f = fa['Wo'].reshape(NH, HD, D)

    normed_hidden = pl.pallas_call(
        functools.partial(_fattn_kernel, NH=NH, HD=HD, QBLK=QBLK),
        grid=(B, NQ, NH),
        in_specs=[
            pl.BlockSpec((1, QBLK, D), lambda b, q, h: (b, q, 0)),
            pl.BlockSpec((1, 1, S, HD), lambda b, q, h: (b, h, 0, 0)),
            pl.BlockSpec((1, 1, S, HD), lambda b, q, h: (b, h, 0, 0)),
            pl.BlockSpec((1, D, HD), lambda b, q, h: (h, 0, 0)),
            pl.BlockSpec((1, 1, HD), lambda b, q, h: (h, 0, 0)),
            pl.BlockSpec((1, HD, D), lambda b, q, h: (h, 0, 0)),
            pl.BlockSpec((1, D), lambda b, q, h: (0, 0)),
            pl.BlockSpec((1, D), lambda b, q, h: (0, 0)),
            pl.BlockSpec((1, D), lambda b, q, h: (0, 0)),
        ],
        out_specs=pl.BlockSpec((1, QBLK, D), lambda b, q, h: (b, q, 0)),
        out_shape=jax.ShapeDtypeStruct((B, S, D), f32),
        compiler_params=pltpu.CompilerParams(
            dimension_semantics=("parallel", "parallel", "arbitrary")),
    )(hidden_states, kph, vph, wq_f, bq_f, wo_f, _row(fa['bo']),
      _row(p['g_f']), _row(p['b_f']))

    sa = p['sattn']
    wq_s = sa['Wq'].reshape(D, NH, HD).transpose(1, 0, 2)
    wk_s = sa['Wk'].reshape(D, NH, HD).transpose(1, 0, 2)
    wv_s = sa['Wv'].reshape(D, NH, HD).transpose(1, 0, 2)
    bq_s = sa['bq'].reshape(NH, 1, HD)
    bk_s = sa['bk'].reshape(NH, 1, HD)
    bv_s = sa['bv'].reshape(NH, 1, HD)
    wo_s = sa['Wo'].reshape(NH, HD, D)
    out = pl.pallas_call(
        functools.partial(_sattn_kernel, NH=NH, HD=HD),
        grid=(B, NQ, NH),
        in_specs=[
            pl.BlockSpec((1, QBLK, D), lambda b, q, h: (b, q, 0)),
            pl.BlockSpec((1, F, D), lambda b, q, h: (b, 0, 0)),
            pl.BlockSpec((1, D, HD), lambda b, q, h: (h, 0, 0)),
            pl.BlockSpec((1, 1, HD), lambda b, q, h: (h, 0, 0)),
            pl.BlockSpec((1, D, HD), lambda b, q, h: (h, 0, 0)),
            pl.BlockSpec((1, 1, HD), lambda b, q, h: (h, 0, 0)),
            pl.BlockSpec((1, D, HD), lambda b, q, h: (h, 0, 0)),
            pl.BlockSpec((1, 1, HD), lambda b, q, h: (h, 0, 0)),
            pl.BlockSpec((1, HD, D), lambda b, q, h: (h, 0, 0)),
            pl.BlockSpec((1, D), lambda b, q, h: (0, 0)),
            pl.BlockSpec((1, D), lambda b, q, h: (0, 0)),
            pl.BlockSpec((1, D), lambda b, q, h: (0, 0)),
        ],
        out_specs=pl.BlockSpec((1, QBLK, D), lambda b, q, h: (b, q, 0)),
        out_shape=jax.ShapeDtypeStruct((B, S, D), f32),
        compiler_params=pltpu.CompilerParams(
            dimension_semantics=("parallel", "parallel", "arbitrary")),
    )(normed_hidden, feats, wq_s, bq_s, wk_s, bk_s, wv_s, bv_s, wo_s,
      _row(sa['bo']), _row(p['g_o']), _row(p['b_o']))

    return (out, jnp.zeros_like(out), jnp.zeros_like(out))


# speedup vs baseline: 1.3008x; 1.3008x over previous
import functools

import jax
import jax.numpy as jnp
import numpy as np
from jax.experimental import pallas as pl
from jax.experimental.pallas import tpu as pltpu

NEG = -1e9
EPS = 1e-12


def _ln(x, g, b):
    m = jnp.mean(x, axis=-1, keepdims=True)
    v = jnp.mean((x - m) ** 2, axis=-1, keepdims=True)
    return (x - m) / jnp.sqrt(v + EPS) * g + b


def _dot(a, b):
    return jax.lax.dot_general(a, b, (((1,), (0,)), ((), ())),
                               preferred_element_type=jnp.float32)


def _dot_t(a, b):
    return jax.lax.dot_general(a, b, (((1,), (1,)), ((), ())),
                               preferred_element_type=jnp.float32)


def _moe_kernel(tag_ref, wm_ref, bm_ref, wg_ref, we_ref, be_ref, out_ref, *, E, DP):
    x = _dot(tag_ref[...], wm_ref[...]) + bm_ref[...]
    logits = _dot(x, wg_ref[...])
    iota = jax.lax.broadcasted_iota(jnp.int32, logits.shape, 1)
    m1 = jnp.max(logits, axis=-1, keepdims=True)
    i1 = jnp.min(jnp.where(logits == m1, iota, E), axis=-1, keepdims=True)
    oh1 = (iota == i1)
    masked = jnp.where(oh1, -1e30, logits)
    m2 = jnp.max(masked, axis=-1, keepdims=True)
    i2 = jnp.min(jnp.where(masked == m2, iota, E), axis=-1, keepdims=True)
    oh2 = (iota == i2)
    e2 = jnp.exp(m2 - m1)
    g1 = 1.0 / (1.0 + e2)
    g2 = e2 / (1.0 + e2)
    w = g1 * oh1.astype(jnp.float32) + g2 * oh2.astype(jnp.float32)
    eo = jnp.maximum(_dot(x, we_ref[...]) + be_ref[...], 0.0)
    acc = x
    for e in range(E):
        acc = acc + w[:, e:e + 1] * eo[:, e * DP:(e + 1) * DP]
    out_ref[...] = acc


def _pqkv_kernel(pr_ref, wq_ref, bq_ref, wk_ref, bk_ref, wv_ref, bv_ref,
                 gpr_ref, bpr_ref, q_ref, k_ref, v_ref):
    pr = pr_ref[0]
    normed = _ln(pr, gpr_ref[...], bpr_ref[...])
    q_ref[0] = _dot(normed, wq_ref[...]) + bq_ref[...]
    k_ref[0] = _dot(normed, wk_ref[...]) + bk_ref[...]
    v_ref[0] = _dot(normed, wv_ref[...]) + bv_ref[...]


def _pattn_kernel(qh_ref, kh_ref, vh_ref, pr_ref,
                  wo_ref, bo_ref, wp_ref, bp_ref, gh_ref, bh_ref,
                  wkf_ref, bkf_ref, wvf_ref, bvf_ref,
                  kf_ref, vf_ref, *, NH, HD, QBLK, KBLK):
    qi = pl.program_id(1)
    qh = qh_ref[0]
    DP = qh.shape[1]
    q0 = qi * QBLK

    lane = jax.lax.broadcasted_iota(jnp.int32, (1, DP), 1)
    row = jax.lax.broadcasted_iota(jnp.int32, (QBLK, KBLK), 0) + q0
    col0 = jax.lax.broadcasted_iota(jnp.int32, (QBLK, KBLK), 1)
    scale = 1.0 / np.sqrt(float(HD))
    nkb = qi * QBLK // KBLK + 1

    def body_h(h, total):
        hm = (lane // HD == h).astype(jnp.float32)
        qm = qh * hm

        def body_k(kb, carry):
            m, l, acc = carry
            k = kh_ref[0, pl.ds(kb * KBLK, KBLK), :]
            v = vh_ref[0, pl.ds(kb * KBLK, KBLK), :] * hm
            s = _dot_t(qm, k) * scale
            s = jnp.where(col0 + kb * KBLK > row, NEG, s)
            m_new = jnp.maximum(m, jnp.max(s, axis=-1, keepdims=True))
            alpha = jnp.exp(m - m_new)
            e = jnp.exp(s - m_new)
            l_new = l * alpha + jnp.sum(e, axis=-1, keepdims=True)
            acc_new = acc * alpha + _dot(e, v)
            return m_new, l_new, acc_new

        m0 = jnp.full((QBLK, 1), -1e30, jnp.float32)
        l0 = jnp.zeros((QBLK, 1), jnp.float32)
        a0 = jnp.zeros((QBLK, DP), jnp.float32)
        m, l, acc = jax.lax.fori_loop(0, nkb, body_k, (m0, l0, a0))
        return total + acc / l

    acc = jax.lax.fori_loop(0, NH, body_h,
                            jnp.zeros((QBLK, DP), jnp.float32))
    o = _dot(acc, wo_ref[...]) + bo_ref[...] + pr_ref[0]
    pr768 = _dot(o, wp_ref[...]) + bp_ref[...]
    normed = _ln(pr768, gh_ref[...], bh_ref[...])
    kf = _dot(normed, wkf_ref[...]) + bkf_ref[...]
    vf = _dot(normed, wvf_ref[...]) + bvf_ref[...]
    HDF = kf.shape[1] // NH
    for h in range(NH):
        kf_ref[0, h] = kf[:, h * HDF:(h + 1) * HDF]
        vf_ref[0, h] = vf[:, h * HDF:(h + 1) * HDF]


def _fattn_kernel(hid_ref, k_ref, v_ref, wq_ref, bq_ref, wo_ref, bo_ref,
                  gf_ref, bf_ref, out_ref, *, NH, HD, QBLK, KBLK):
    qi = pl.program_id(1)
    hid = hid_ref[0]
    qall = _dot(hid, wq_ref[...]) + bq_ref[...]
    scale = 1.0 / np.sqrt(float(HD))
    row = jax.lax.broadcasted_iota(jnp.int32, (QBLK, KBLK), 0) + qi * QBLK
    col0 = jax.lax.broadcasted_iota(jnp.int32, (QBLK, KBLK), 1)
    nkb = qi * QBLK // KBLK + 1

    parts = []
    for h in range(NH):
        q = qall[:, h * HD:(h + 1) * HD]

        def body_k(kb, carry, h=h, q=q):
            m, l, acc = carry
            k = k_ref[0, h, pl.ds(kb * KBLK, KBLK), :]
            v = v_ref[0, h, pl.ds(kb * KBLK, KBLK), :]
            s = _dot_t(q, k) * scale
            s = jnp.where(col0 + kb * KBLK > row, NEG, s)
            m_new = jnp.maximum(m, jnp.max(s, axis=-1, keepdims=True))
            alpha = jnp.exp(m - m_new)
            e = jnp.exp(s - m_new)
            l_new = l * alpha + jnp.sum(e, axis=-1, keepdims=True)
            acc_new = acc * alpha + _dot(e, v)
            return m_new, l_new, acc_new

        m0 = jnp.full((QBLK, 1), -1e30, jnp.float32)
        l0 = jnp.zeros((QBLK, 1), jnp.float32)
        a0 = jnp.zeros((QBLK, HD), jnp.float32)
        m, l, acc = jax.lax.fori_loop(0, nkb, body_k, (m0, l0, a0))
        parts.append(acc / l)
    o = jnp.concatenate(parts, axis=1)
    out = _dot(o, wo_ref[...]) + bo_ref[...] + hid
    out_ref[0] = _ln(out, gf_ref[...], bf_ref[...])


def _sattn_kernel(x_ref, f_ref, wq_ref, bq_ref, wk_ref, bk_ref, wv_ref,
                  bv_ref, wo_ref, bo_ref, go_ref, bo2_ref, out_ref,
                  *, NH, HD):
    x = x_ref[0]
    f = f_ref[0]
    qall = _dot(x, wq_ref[...]) + bq_ref[...]
    kall = _dot(f, wk_ref[...]) + bk_ref[...]
    vall = _dot(f, wv_ref[...]) + bv_ref[...]
    scale = 1.0 / np.sqrt(float(HD))
    parts = []
    for h in range(NH):
        q = qall[:, h * HD:(h + 1) * HD]
        k = kall[:, h * HD:(h + 1) * HD]
        v = vall[:, h * HD:(h + 1) * HD]
        s = _dot_t(q, k) * scale
        m = jnp.max(s, axis=-1, keepdims=True)
        e = jnp.exp(s - m)
        p = e / jnp.sum(e, axis=-1, keepdims=True)
        parts.append(_dot(p, v))
    o = jnp.concatenate(parts, axis=1)
    out = _dot(o, wo_ref[...]) + bo_ref[...] + x
    out_ref[0] = _ln(out, go_ref[...], bo2_ref[...])


def _row(x):
    return x.reshape(1, -1)


def kernel(hidden_states, tag_hidden_state, feats, video_ids, params):
    p = params
    B, S, D = hidden_states.shape
    TAG_D = tag_hidden_state.shape[2]
    DP = p['Wm'].shape[1]
    E = p['Wg'].shape[1]
    NH = D // 64
    HD = D // NH
    HDP = DP // NH
    F = feats.shape[1]
    QBLK = 256
    NQ = S // QBLK
    MBLK = 512
    NM = (B * S) // MBLK

    f32 = jnp.float32

    tag2d = tag_hidden_state.reshape(B * S, TAG_D)
    we2 = p['We'].transpose(1, 0, 2).reshape(DP, E * DP)
    be2 = p['be'].reshape(1, E * DP)
    priori = pl.pallas_call(
        functools.partial(_moe_kernel, E=E, DP=DP),
        grid=(NM,),
        in_specs=[
            pl.BlockSpec((MBLK, TAG_D), lambda i: (i, 0)),
            pl.BlockSpec((TAG_D, DP), lambda i: (0, 0)),
            pl.BlockSpec((1, DP), lambda i: (0, 0)),
            pl.BlockSpec((DP, E), lambda i: (0, 0)),
            pl.BlockSpec((DP, E * DP), lambda i: (0, 0)),
            pl.BlockSpec((1, E * DP), lambda i: (0, 0)),
        ],
        out_specs=pl.BlockSpec((MBLK, DP), lambda i: (i, 0)),
        out_shape=jax.ShapeDtypeStruct((B * S, DP), f32),
    )(tag2d, p['Wm'], _row(p['bm']), p['Wg'], we2, be2)
    priori = priori.reshape(B, S, DP)

    pa = p['pattn']
    pqh, pkh, pvh = pl.pallas_call(
        _pqkv_kernel,
        grid=(B, NQ),
        in_specs=[
            pl.BlockSpec((1, QBLK, DP), lambda b, q: (b, q, 0)),
            pl.BlockSpec((DP, DP), lambda b, q: (0, 0)),
            pl.BlockSpec((1, DP), lambda b, q: (0, 0)),
            pl.BlockSpec((DP, DP), lambda b, q: (0, 0)),
            pl.BlockSpec((1, DP), lambda b, q: (0, 0)),
            pl.BlockSpec((DP, DP), lambda b, q: (0, 0)),
            pl.BlockSpec((1, DP), lambda b, q: (0, 0)),
            pl.BlockSpec((1, DP), lambda b, q: (0, 0)),
            pl.BlockSpec((1, DP), lambda b, q: (0, 0)),
        ],
        out_specs=[pl.BlockSpec((1, QBLK, DP), lambda b, q: (b, q, 0))] * 3,
        out_shape=[jax.ShapeDtypeStruct((B, S, DP), f32)] * 3,
    )(priori, pa['Wq'], _row(pa['bq']), pa['Wk'], _row(pa['bk']),
      pa['Wv'], _row(pa['bv']), _row(p['g_pr']), _row(p['b_pr']))

    fa = p['fattn']
    kp, vp = pl.pallas_call(
        functools.partial(_pattn_kernel, NH=NH, HD=HDP, QBLK=QBLK, KBLK=QBLK),
        grid=(B, NQ),
        in_specs=[
            pl.BlockSpec((1, QBLK, DP), lambda b, q: (b, q, 0)),
            pl.BlockSpec((1, S, DP), lambda b, q: (b, 0, 0)),
            pl.BlockSpec((1, S, DP), lambda b, q: (b, 0, 0)),
            pl.BlockSpec((1, QBLK, DP), lambda b, q: (b, q, 0)),
            pl.BlockSpec((DP, DP), lambda b, q: (0, 0)),
            pl.BlockSpec((1, DP), lambda b, q: (0, 0)),
            pl.BlockSpec((DP, D), lambda b, q: (0, 0)),
            pl.BlockSpec((1, D), lambda b, q: (0, 0)),
            pl.BlockSpec((1, D), lambda b, q: (0, 0)),
            pl.BlockSpec((1, D), lambda b, q: (0, 0)),
            pl.BlockSpec((D, D), lambda b, q: (0, 0)),
            pl.BlockSpec((1, D), lambda b, q: (0, 0)),
            pl.BlockSpec((D, D), lambda b, q: (0, 0)),
            pl.BlockSpec((1, D), lambda b, q: (0, 0)),
        ],
        out_specs=[pl.BlockSpec((1, NH, QBLK, HD),
                                lambda b, q: (b, 0, q, 0))] * 2,
        out_shape=[jax.ShapeDtypeStruct((B, NH, S, HD), f32)] * 2,
    )(pqh, pkh, pvh, priori, pa['Wo'], _row(pa['bo']),
      p['Wp'], _row(p['bp']), _row(p['g_h']), _row(p['b_h']),
      fa['Wk'], _row(fa['bk']), fa['Wv'], _row(fa['bv']))
    kph, vph = kp, vp

    normed_hidden = pl.pallas_call(
        functools.partial(_fattn_kernel, NH=NH, HD=HD, QBLK=QBLK, KBLK=512),
        grid=(B, NQ),
        in_specs=[
            pl.BlockSpec((1, QBLK, D), lambda b, q: (b, q, 0)),
            pl.BlockSpec((1, NH, S, HD), lambda b, q: (b, 0, 0, 0)),
            pl.BlockSpec((1, NH, S, HD), lambda b, q: (b, 0, 0, 0)),
            pl.BlockSpec((D, D), lambda b, q: (0, 0)),
            pl.BlockSpec((1, D), lambda b, q: (0, 0)),
            pl.BlockSpec((D, D), lambda b, q: (0, 0)),
            pl.BlockSpec((1, D), lambda b, q: (0, 0)),
            pl.BlockSpec((1, D), lambda b, q: (0, 0)),
            pl.BlockSpec((1, D), lambda b, q: (0, 0)),
        ],
        out_specs=pl.BlockSpec((1, QBLK, D), lambda b, q: (b, q, 0)),
        out_shape=jax.ShapeDtypeStruct((B, S, D), f32),
        compiler_params=pltpu.CompilerParams(
            dimension_semantics=("parallel", "arbitrary"),
            vmem_limit_bytes=100 * 1024 * 1024),
    )(hidden_states, kph, vph, fa['Wq'], _row(fa['bq']),
      fa['Wo'], _row(fa['bo']), _row(p['g_f']), _row(p['b_f']))

    sa = p['sattn']
    out = pl.pallas_call(
        functools.partial(_sattn_kernel, NH=NH, HD=HD),
        grid=(B, NQ),
        in_specs=[
            pl.BlockSpec((1, QBLK, D), lambda b, q: (b, q, 0)),
            pl.BlockSpec((1, F, D), lambda b, q: (b, 0, 0)),
            pl.BlockSpec((D, D), lambda b, q: (0, 0)),
            pl.BlockSpec((1, D), lambda b, q: (0, 0)),
            pl.BlockSpec((D, D), lambda b, q: (0, 0)),
            pl.BlockSpec((1, D), lambda b, q: (0, 0)),
            pl.BlockSpec((D, D), lambda b, q: (0, 0)),
            pl.BlockSpec((1, D), lambda b, q: (0, 0)),
            pl.BlockSpec((D, D), lambda b, q: (0, 0)),
            pl.BlockSpec((1, D), lambda b, q: (0, 0)),
            pl.BlockSpec((1, D), lambda b, q: (0, 0)),
            pl.BlockSpec((1, D), lambda b, q: (0, 0)),
        ],
        out_specs=pl.BlockSpec((1, QBLK, D), lambda b, q: (b, q, 0)),
        out_shape=jax.ShapeDtypeStruct((B, S, D), f32),
    )(normed_hidden, feats, sa['Wq'], _row(sa['bq']), sa['Wk'], _row(sa['bk']),
      sa['Wv'], _row(sa['bv']), sa['Wo'], _row(sa['bo']),
      _row(p['g_o']), _row(p['b_o']))

    return (out, jnp.zeros_like(out), jnp.zeros_like(out))


# speedup vs baseline: 1.5165x; 1.1658x over previous
import functools

import jax
import jax.numpy as jnp
import numpy as np
from jax.experimental import pallas as pl
from jax.experimental.pallas import tpu as pltpu

NEG = -1e9
EPS = 1e-12


def _ln(x, g, b):
    m = jnp.mean(x, axis=-1, keepdims=True)
    v = jnp.mean((x - m) ** 2, axis=-1, keepdims=True)
    return (x - m) / jnp.sqrt(v + EPS) * g + b


def _dot(a, b):
    return jax.lax.dot_general(a, b, (((1,), (0,)), ((), ())),
                               preferred_element_type=jnp.float32)


def _dot_t(a, b):
    return jax.lax.dot_general(a, b, (((1,), (1,)), ((), ())),
                               preferred_element_type=jnp.float32)


def _bf(x):
    return x.astype(jnp.bfloat16)


def _dotb(a, b):
    return jax.lax.dot_general(_bf(a), _bf(b), (((1,), (0,)), ((), ())),
                               preferred_element_type=jnp.float32)


def _dotb_t(a, b):
    return jax.lax.dot_general(_bf(a), _bf(b), (((1,), (1,)), ((), ())),
                               preferred_element_type=jnp.float32)


def _moe_kernel(tag_ref, wm_ref, bm_ref, wg_ref, we_ref, be_ref, out_ref, *, E, DP):
    x = _dot(tag_ref[...], wm_ref[...]) + bm_ref[...]
    logits = _dot(x, wg_ref[...])
    iota = jax.lax.broadcasted_iota(jnp.int32, logits.shape, 1)
    m1 = jnp.max(logits, axis=-1, keepdims=True)
    i1 = jnp.min(jnp.where(logits == m1, iota, E), axis=-1, keepdims=True)
    oh1 = (iota == i1)
    masked = jnp.where(oh1, -1e30, logits)
    m2 = jnp.max(masked, axis=-1, keepdims=True)
    i2 = jnp.min(jnp.where(masked == m2, iota, E), axis=-1, keepdims=True)
    oh2 = (iota == i2)
    e2 = jnp.exp(m2 - m1)
    g1 = 1.0 / (1.0 + e2)
    g2 = e2 / (1.0 + e2)
    w = g1 * oh1.astype(jnp.float32) + g2 * oh2.astype(jnp.float32)
    eo = jnp.maximum(_dotb(x, we_ref[...]) + be_ref[...], 0.0)
    acc = x
    for e in range(E):
        acc = acc + w[:, e:e + 1] * eo[:, e * DP:(e + 1) * DP]
    out_ref[...] = acc


def _pqkv_kernel(pr_ref, wq_ref, bq_ref, wk_ref, bk_ref, wv_ref, bv_ref,
                 gpr_ref, bpr_ref, q_ref, k_ref, v_ref):
    pr = pr_ref[0]
    normed = _ln(pr, gpr_ref[...], bpr_ref[...])
    q_ref[0] = _bf(_dotb(normed, wq_ref[...]) + bq_ref[...])
    k_ref[0] = _bf(_dotb(normed, wk_ref[...]) + bk_ref[...])
    v_ref[0] = _bf(_dotb(normed, wv_ref[...]) + bv_ref[...])


def _pattn_kernel(qh_ref, kh_ref, vh_ref, pr_ref,
                  wo_ref, bo_ref, wp_ref, bp_ref, gh_ref, bh_ref,
                  wkf_ref, bkf_ref, wvf_ref, bvf_ref,
                  kf_ref, vf_ref, *, NH, HD, QBLK, KBLK):
    qi = pl.program_id(1)
    qh = qh_ref[0]
    DP = qh.shape[1]
    q0 = qi * QBLK

    lane = jax.lax.broadcasted_iota(jnp.int32, (1, DP), 1)
    row = jax.lax.broadcasted_iota(jnp.int32, (QBLK, KBLK), 0) + q0
    col0 = jax.lax.broadcasted_iota(jnp.int32, (QBLK, KBLK), 1)
    scale = 1.0 / np.sqrt(float(HD))
    nkb = qi * QBLK // KBLK + 1

    nfull = q0 // KBLK

    def body_h(h, total):
        hm = (lane // HD == h)
        qm = qh * hm.astype(jnp.bfloat16)

        def body_nm(kb, carry):
            l, acc = carry
            k = kh_ref[0, pl.ds(kb * KBLK, KBLK), :]
            v = vh_ref[0, pl.ds(kb * KBLK, KBLK), :]
            e = jnp.exp(_dot_t(qm, k) * scale)
            return l + jnp.sum(e, axis=-1, keepdims=True), acc + _dot(_bf(e), v)

        def body_m(kb, carry):
            l, acc = carry
            k = kh_ref[0, pl.ds(kb * KBLK, KBLK), :]
            v = vh_ref[0, pl.ds(kb * KBLK, KBLK), :]
            s = _dot_t(qm, k) * scale
            s = jnp.where(col0 + kb * KBLK > row, NEG, s)
            e = jnp.exp(s)
            return l + jnp.sum(e, axis=-1, keepdims=True), acc + _dot(_bf(e), v)

        l0 = jnp.zeros((QBLK, 1), jnp.float32)
        a0 = jnp.zeros((QBLK, DP), jnp.float32)
        l, acc = jax.lax.fori_loop(0, nfull, body_nm, (l0, a0))
        l, acc = jax.lax.fori_loop(nfull, nkb, body_m, (l, acc))
        return total + jnp.where(hm, acc, 0.0) / l

    acc = jax.lax.fori_loop(0, NH, body_h,
                            jnp.zeros((QBLK, DP), jnp.float32))
    o = _dotb(acc, wo_ref[...]) + bo_ref[...] + pr_ref[0]
    pr768 = _dotb(o, wp_ref[...]) + bp_ref[...]
    normed = _ln(pr768, gh_ref[...], bh_ref[...])
    kf = _dotb(normed, wkf_ref[...]) + bkf_ref[...]
    vf = _dotb(normed, wvf_ref[...]) + bvf_ref[...]
    HDF = kf.shape[1] // NH
    for h in range(NH):
        kf_ref[0, h] = _bf(kf[:, h * HDF:(h + 1) * HDF])
        vf_ref[0, h] = _bf(vf[:, h * HDF:(h + 1) * HDF])


def _fattn_kernel(hid_ref, k_ref, v_ref, wq_ref, bq_ref, wo_ref, bo_ref,
                  gf_ref, bf_ref, out_ref, *, NH, HD, QBLK, KBLK):
    qi = pl.program_id(1)
    hid = hid_ref[0]
    qall = _bf(_dotb(hid, wq_ref[...]) + bq_ref[...])
    scale = 1.0 / np.sqrt(float(HD))
    row = jax.lax.broadcasted_iota(jnp.int32, (QBLK, KBLK), 0) + qi * QBLK
    col0 = jax.lax.broadcasted_iota(jnp.int32, (QBLK, KBLK), 1)
    nkb = qi * QBLK // KBLK + 1

    nfull = qi * QBLK // KBLK
    parts = []
    for h in range(NH):
        q = qall[:, h * HD:(h + 1) * HD]

        def body_nm(kb, carry, h=h, q=q):
            l, acc = carry
            k = k_ref[0, h, pl.ds(kb * KBLK, KBLK), :]
            v = v_ref[0, h, pl.ds(kb * KBLK, KBLK), :]
            e = jnp.exp(_dot_t(q, k) * scale)
            return l + jnp.sum(e, axis=-1, keepdims=True), acc + _dot(_bf(e), v)

        def body_m(kb, carry, h=h, q=q):
            l, acc = carry
            k = k_ref[0, h, pl.ds(kb * KBLK, KBLK), :]
            v = v_ref[0, h, pl.ds(kb * KBLK, KBLK), :]
            s = _dot_t(q, k) * scale
            s = jnp.where(col0 + kb * KBLK > row, NEG, s)
            e = jnp.exp(s)
            return l + jnp.sum(e, axis=-1, keepdims=True), acc + _dot(_bf(e), v)

        l0 = jnp.zeros((QBLK, 1), jnp.float32)
        a0 = jnp.zeros((QBLK, HD), jnp.float32)
        l, acc = jax.lax.fori_loop(0, nfull, body_nm, (l0, a0))
        l, acc = jax.lax.fori_loop(nfull, nkb, body_m, (l, acc))
        parts.append(acc / l)
    o = jnp.concatenate(parts, axis=1)
    out = _dotb(o, wo_ref[...]) + bo_ref[...] + hid
    out_ref[0] = _ln(out, gf_ref[...], bf_ref[...])


def _sattn_kernel(x_ref, f_ref, wq_ref, bq_ref, wk_ref, bk_ref, wv_ref,
                  bv_ref, wo_ref, bo_ref, go_ref, bo2_ref, out_ref,
                  *, NH, HD):
    x = x_ref[0]
    f = f_ref[0]
    qall = _bf(_dotb(x, wq_ref[...]) + bq_ref[...])
    kall = _bf(_dotb(f, wk_ref[...]) + bk_ref[...])
    vall = _bf(_dotb(f, wv_ref[...]) + bv_ref[...])
    scale = 1.0 / np.sqrt(float(HD))
    parts = []
    for h in range(NH):
        q = qall[:, h * HD:(h + 1) * HD]
        k = kall[:, h * HD:(h + 1) * HD]
        v = vall[:, h * HD:(h + 1) * HD]
        e = jnp.exp(_dot_t(q, k) * scale)
        p = e / jnp.sum(e, axis=-1, keepdims=True)
        parts.append(_dot(_bf(p), v))
    o = jnp.concatenate(parts, axis=1)
    out = _dotb(o, wo_ref[...]) + bo_ref[...] + x
    out_ref[0] = _ln(out, go_ref[...], bo2_ref[...])


def _row(x):
    return x.reshape(1, -1)


def kernel(hidden_states, tag_hidden_state, feats, video_ids, params):
    p = params
    B, S, D = hidden_states.shape
    TAG_D = tag_hidden_state.shape[2]
    DP = p['Wm'].shape[1]
    E = p['Wg'].shape[1]
    NH = D // 64
    HD = D // NH
    HDP = DP // NH
    F = feats.shape[1]
    QBLK = 256
    NQ = S // QBLK
    MBLK = 512
    NM = (B * S) // MBLK

    f32 = jnp.float32

    tag2d = tag_hidden_state.reshape(B * S, TAG_D)
    we2 = p['We'].transpose(1, 0, 2).reshape(DP, E * DP)
    be2 = p['be'].reshape(1, E * DP)
    priori = pl.pallas_call(
        functools.partial(_moe_kernel, E=E, DP=DP),
        grid=(NM,),
        in_specs=[
            pl.BlockSpec((MBLK, TAG_D), lambda i: (i, 0)),
            pl.BlockSpec((TAG_D, DP), lambda i: (0, 0)),
            pl.BlockSpec((1, DP), lambda i: (0, 0)),
            pl.BlockSpec((DP, E), lambda i: (0, 0)),
            pl.BlockSpec((DP, E * DP), lambda i: (0, 0)),
            pl.BlockSpec((1, E * DP), lambda i: (0, 0)),
        ],
        out_specs=pl.BlockSpec((MBLK, DP), lambda i: (i, 0)),
        out_shape=jax.ShapeDtypeStruct((B * S, DP), f32),
    )(tag2d, p['Wm'], _row(p['bm']), p['Wg'], we2, be2)
    priori = priori.reshape(B, S, DP)

    pa = p['pattn']
    pqh, pkh, pvh = pl.pallas_call(
        _pqkv_kernel,
        grid=(B, NQ),
        in_specs=[
            pl.BlockSpec((1, QBLK, DP), lambda b, q: (b, q, 0)),
            pl.BlockSpec((DP, DP), lambda b, q: (0, 0)),
            pl.BlockSpec((1, DP), lambda b, q: (0, 0)),
            pl.BlockSpec((DP, DP), lambda b, q: (0, 0)),
            pl.BlockSpec((1, DP), lambda b, q: (0, 0)),
            pl.BlockSpec((DP, DP), lambda b, q: (0, 0)),
            pl.BlockSpec((1, DP), lambda b, q: (0, 0)),
            pl.BlockSpec((1, DP), lambda b, q: (0, 0)),
            pl.BlockSpec((1, DP), lambda b, q: (0, 0)),
        ],
        out_specs=[pl.BlockSpec((1, QBLK, DP), lambda b, q: (b, q, 0))] * 3,
        out_shape=[jax.ShapeDtypeStruct((B, S, DP), jnp.bfloat16)] * 3,
    )(priori, pa['Wq'], _row(pa['bq']), pa['Wk'], _row(pa['bk']),
      pa['Wv'], _row(pa['bv']), _row(p['g_pr']), _row(p['b_pr']))

    fa = p['fattn']
    kp, vp = pl.pallas_call(
        functools.partial(_pattn_kernel, NH=NH, HD=HDP, QBLK=QBLK, KBLK=QBLK),
        grid=(B, NQ),
        in_specs=[
            pl.BlockSpec((1, QBLK, DP), lambda b, q: (b, q, 0)),
            pl.BlockSpec((1, S, DP), lambda b, q: (b, 0, 0)),
            pl.BlockSpec((1, S, DP), lambda b, q: (b, 0, 0)),
            pl.BlockSpec((1, QBLK, DP), lambda b, q: (b, q, 0)),
            pl.BlockSpec((DP, DP), lambda b, q: (0, 0)),
            pl.BlockSpec((1, DP), lambda b, q: (0, 0)),
            pl.BlockSpec((DP, D), lambda b, q: (0, 0)),
            pl.BlockSpec((1, D), lambda b, q: (0, 0)),
            pl.BlockSpec((1, D), lambda b, q: (0, 0)),
            pl.BlockSpec((1, D), lambda b, q: (0, 0)),
            pl.BlockSpec((D, D), lambda b, q: (0, 0)),
            pl.BlockSpec((1, D), lambda b, q: (0, 0)),
            pl.BlockSpec((D, D), lambda b, q: (0, 0)),
            pl.BlockSpec((1, D), lambda b, q: (0, 0)),
        ],
        out_specs=[pl.BlockSpec((1, NH, QBLK, HD),
                                lambda b, q: (b, 0, q, 0))] * 2,
        out_shape=[jax.ShapeDtypeStruct((B, NH, S, HD), jnp.bfloat16)] * 2,
    )(pqh, pkh, pvh, priori, pa['Wo'], _row(pa['bo']),
      p['Wp'], _row(p['bp']), _row(p['g_h']), _row(p['b_h']),
      fa['Wk'], _row(fa['bk']), fa['Wv'], _row(fa['bv']))
    kph, vph = kp, vp

    normed_hidden = pl.pallas_call(
        functools.partial(_fattn_kernel, NH=NH, HD=HD, QBLK=QBLK, KBLK=512),
        grid=(B, NQ),
        in_specs=[
            pl.BlockSpec((1, QBLK, D), lambda b, q: (b, q, 0)),
            pl.BlockSpec((1, NH, S, HD), lambda b, q: (b, 0, 0, 0)),
            pl.BlockSpec((1, NH, S, HD), lambda b, q: (b, 0, 0, 0)),
            pl.BlockSpec((D, D), lambda b, q: (0, 0)),
            pl.BlockSpec((1, D), lambda b, q: (0, 0)),
            pl.BlockSpec((D, D), lambda b, q: (0, 0)),
            pl.BlockSpec((1, D), lambda b, q: (0, 0)),
            pl.BlockSpec((1, D), lambda b, q: (0, 0)),
            pl.BlockSpec((1, D), lambda b, q: (0, 0)),
        ],
        out_specs=pl.BlockSpec((1, QBLK, D), lambda b, q: (b, q, 0)),
        out_shape=jax.ShapeDtypeStruct((B, S, D), f32),
        compiler_params=pltpu.CompilerParams(
            dimension_semantics=("parallel", "arbitrary"),
            vmem_limit_bytes=100 * 1024 * 1024),
    )(hidden_states, kph, vph, fa['Wq'], _row(fa['bq']),
      fa['Wo'], _row(fa['bo']), _row(p['g_f']), _row(p['b_f']))

    sa = p['sattn']
    out = pl.pallas_call(
        functools.partial(_sattn_kernel, NH=NH, HD=HD),
        grid=(B, NQ),
        in_specs=[
            pl.BlockSpec((1, QBLK, D), lambda b, q: (b, q, 0)),
            pl.BlockSpec((1, F, D), lambda b, q: (b, 0, 0)),
            pl.BlockSpec((D, D), lambda b, q: (0, 0)),
            pl.BlockSpec((1, D), lambda b, q: (0, 0)),
            pl.BlockSpec((D, D), lambda b, q: (0, 0)),
            pl.BlockSpec((1, D), lambda b, q: (0, 0)),
            pl.BlockSpec((D, D), lambda b, q: (0, 0)),
            pl.BlockSpec((1, D), lambda b, q: (0, 0)),
            pl.BlockSpec((D, D), lambda b, q: (0, 0)),
            pl.BlockSpec((1, D), lambda b, q: (0, 0)),
            pl.BlockSpec((1, D), lambda b, q: (0, 0)),
            pl.BlockSpec((1, D), lambda b, q: (0, 0)),
        ],
        out_specs=pl.BlockSpec((1, QBLK, D), lambda b, q: (b, q, 0)),
        out_shape=jax.ShapeDtypeStruct((B, S, D), f32),
    )(normed_hidden, feats, sa['Wq'], _row(sa['bq']), sa['Wk'], _row(sa['bk']),
      sa['Wv'], _row(sa['bv']), sa['Wo'], _row(sa['bo']),
      _row(p['g_o']), _row(p['b_o']))

    return (out, jnp.zeros_like(out), jnp.zeros_like(out))
